# Initial kernel scaffold; baseline (speedup 1.0000x reference)
import jax
import jax.numpy as jnp
from jax.experimental import pallas as pl


def kernel(inp, sg):
    raise NotImplementedError("write your pallas kernel here")


# speedup vs baseline: 1.5156x; 1.5156x over previous
import jax
import jax.numpy as jnp
import numpy as np
from jax.experimental import pallas as pl
from jax.experimental.pallas import tpu as pltpu

_SIZE = 128
_LO = -0.0001
_HI = 1.0001
_STEP = (_HI - _LO) / _SIZE

_ROWS = 128


def _rbf_body(sg_ref, pts_ref, out_ref):
    s = sg_ref[0]
    inv = -0.5 / (s * s)
    norm = 1.0 / (2.0 * np.float32(np.pi) * s * s)

    x = pts_ref[:, 0:1]
    p = pts_ref[:, 1:2] - x

    ci = _LO + _STEP * jax.lax.broadcasted_iota(
        jnp.int32, (1, _SIZE), 1).astype(jnp.float32)

    gx = jnp.exp((x - ci) * (x - ci) * inv)
    gy = jnp.exp((p - ci) * (p - ci) * inv) * norm

    out_ref[...] = gy[:, :, None] * gx[:, None, :]


def kernel(inp, sg):
    B, P, _ = inp.shape
    n = B * P
    pts = inp.reshape(n, 2)
    out = pl.pallas_call(
        _rbf_body,
        out_shape=jax.ShapeDtypeStruct((n, _SIZE, _SIZE), jnp.float32),
        grid=(n // _ROWS,),
        in_specs=[
            pl.BlockSpec(memory_space=pltpu.SMEM),
            pl.BlockSpec((_ROWS, 2), lambda i: (i, 0)),
        ],
        out_specs=pl.BlockSpec((_ROWS, _SIZE, _SIZE), lambda i: (i, 0, 0)),
        compiler_params=pltpu.CompilerParams(
            dimension_semantics=("parallel",),
        ),
        name="rbf_splat",
    )(sg, pts)
    return out.reshape(B, P, _SIZE * _SIZE)


# speedup vs baseline: 5.6641x; 3.7371x over previous
import jax
import jax.numpy as jnp
import numpy as np
from jax.experimental import pallas as pl
from jax.experimental.pallas import tpu as pltpu

_SIZE = 128
_LO = -0.0001
_HI = 1.0001
_STEP = (_HI - _LO) / _SIZE

_ROWS = 128


def _rbf_body(sg_ref, pts_ref, out_ref):
    s = sg_ref[0]
    inv = -0.5 / (s * s)
    norm = 1.0 / (2.0 * np.float32(np.pi) * s * s)

    x = pts_ref[:, 0:1]
    p = pts_ref[:, 1:2] - x

    ci = _LO + _STEP * jax.lax.broadcasted_iota(
        jnp.int32, (1, _SIZE), 1).astype(jnp.float32)

    gx = jnp.exp((x - ci) * (x - ci) * inv)
    gy = jnp.exp((p - ci) * (p - ci) * inv) * norm

    for j in range(_SIZE):
        out_ref[:, _SIZE * j:_SIZE * (j + 1)] = gx * gy[:, j:j + 1]


def kernel(inp, sg):
    B, P, _ = inp.shape
    n = B * P
    pts = inp.reshape(n, 2)
    out = pl.pallas_call(
        _rbf_body,
        out_shape=jax.ShapeDtypeStruct((n, _SIZE * _SIZE), jnp.float32),
        grid=(n // _ROWS,),
        in_specs=[
            pl.BlockSpec(memory_space=pltpu.SMEM),
            pl.BlockSpec((_ROWS, 2), lambda i: (i, 0)),
        ],
        out_specs=pl.BlockSpec((_ROWS, _SIZE * _SIZE), lambda i: (i, 0)),
        compiler_params=pltpu.CompilerParams(
            dimension_semantics=("parallel",),
        ),
        name="rbf_splat",
    )(sg, pts)
    return out.reshape(B, P, _SIZE * _SIZE)


# speedup vs baseline: 6.0121x; 1.0614x over previous
import jax
import jax.numpy as jnp
import numpy as np
from jax.experimental import pallas as pl
from jax.experimental.pallas import tpu as pltpu

_SIZE = 128
_LO = -0.0001
_HI = 1.0001
_STEP = (_HI - _LO) / _SIZE

_ROWS = 256


def _rbf_body(sg_ref, pts_ref, out_ref):
    s = sg_ref[0]
    inv = -0.5 / (s * s)
    norm = 1.0 / (2.0 * np.float32(np.pi) * s * s)

    x = pts_ref[:, 0:1]
    p = pts_ref[:, 1:2] - x

    ci = _LO + _STEP * jax.lax.broadcasted_iota(
        jnp.int32, (1, _SIZE), 1).astype(jnp.float32)

    gx = jnp.exp((x - ci) * (x - ci) * inv)
    gy = jnp.exp((p - ci) * (p - ci) * inv) * norm

    for j in range(_SIZE):
        out_ref[:, _SIZE * j:_SIZE * (j + 1)] = gx * gy[:, j:j + 1]


def kernel(inp, sg):
    B, P, _ = inp.shape
    n = B * P
    pts = inp.reshape(n, 2)
    out = pl.pallas_call(
        _rbf_body,
        out_shape=jax.ShapeDtypeStruct((n, _SIZE * _SIZE), jnp.float32),
        grid=(n // _ROWS,),
        in_specs=[
            pl.BlockSpec(memory_space=pltpu.SMEM),
            pl.BlockSpec((_ROWS, 2), lambda i: (i, 0)),
        ],
        out_specs=pl.BlockSpec((_ROWS, _SIZE * _SIZE), lambda i: (i, 0)),
        compiler_params=pltpu.CompilerParams(
            dimension_semantics=("parallel",),
            vmem_limit_bytes=56 * 1024 * 1024,
        ),
        name="rbf_splat",
    )(sg, pts)
    return out.reshape(B, P, _SIZE * _SIZE)


# speedup vs baseline: 6.2410x; 1.0381x over previous
import jax
import jax.numpy as jnp
import numpy as np
from jax.experimental import pallas as pl
from jax.experimental.pallas import tpu as pltpu

_SIZE = 128
_LO = -0.0001
_HI = 1.0001
_STEP = (_HI - _LO) / _SIZE

_ROWS = 256


def _rbf_body(sg_ref, pts_ref, out_ref):
    s = sg_ref[0]
    inv = -0.5 / (s * s)
    norm = 1.0 / (2.0 * np.float32(np.pi) * s * s)

    x = pts_ref[:, 0:1]
    p = pts_ref[:, 1:2] - x

    ci = _LO + _STEP * jax.lax.broadcasted_iota(
        jnp.int32, (1, _SIZE), 1).astype(jnp.float32)

    gxn = jnp.exp((x - ci) * (x - ci) * inv) * norm
    gy = jnp.exp((p - ci) * (p - ci) * inv)
    pb = jnp.broadcast_to(p, (_ROWS, _SIZE))

    for j in range(_SIZE):
        sl = slice(_SIZE * j, _SIZE * (j + 1))
        if j % 2 == 0:
            out_ref[:, sl] = gxn * gy[:, j:j + 1]
        else:
            t = pb - (_LO + _STEP * j)
            out_ref[:, sl] = gxn * jnp.exp(t * t * inv)


def kernel(inp, sg):
    B, P, _ = inp.shape
    n = B * P
    pts = inp.reshape(n, 2)
    out = pl.pallas_call(
        _rbf_body,
        out_shape=jax.ShapeDtypeStruct((n, _SIZE * _SIZE), jnp.float32),
        grid=(n // _ROWS,),
        in_specs=[
            pl.BlockSpec(memory_space=pltpu.SMEM),
            pl.BlockSpec((_ROWS, 2), lambda i: (i, 0)),
        ],
        out_specs=pl.BlockSpec((_ROWS, _SIZE * _SIZE), lambda i: (i, 0)),
        compiler_params=pltpu.CompilerParams(
            dimension_semantics=("parallel",),
            vmem_limit_bytes=56 * 1024 * 1024,
        ),
        name="rbf_splat",
    )(sg, pts)
    return out.reshape(B, P, _SIZE * _SIZE)


# speedup vs baseline: 6.3278x; 1.0139x over previous
import jax
import jax.numpy as jnp
import numpy as np
from jax.experimental import pallas as pl
from jax.experimental.pallas import tpu as pltpu

_SIZE = 128
_LO = -0.0001
_HI = 1.0001
_STEP = (_HI - _LO) / _SIZE

_ROWS = 256
_NBLK = 4096 // _ROWS
_NBUF = 3


def _rbf_body(sg_ref, pts_ref, out_hbm, scratch, sems):
    k = pl.program_id(0)
    buf = jax.lax.rem(k, _NBUF)

    s = sg_ref[0]
    inv = -0.5 / (s * s)
    norm = 1.0 / (2.0 * np.float32(np.pi) * s * s)

    x = pts_ref[:, 0:1]
    p = pts_ref[:, 1:2] - x

    ci = _LO + _STEP * jax.lax.broadcasted_iota(
        jnp.int32, (1, _SIZE), 1).astype(jnp.float32)

    gxn = jnp.exp((x - ci) * (x - ci) * inv) * norm
    gy = jnp.exp((p - ci) * (p - ci) * inv)
    pb = jnp.broadcast_to(p, (_ROWS, _SIZE))

    @pl.when(k >= _NBUF)
    def _():
        pltpu.make_async_copy(
            scratch.at[buf],
            out_hbm.at[pl.ds((k - _NBUF) * _ROWS, _ROWS), :],
            sems.at[buf],
        ).wait()

    view = scratch.at[buf]
    for j in range(_SIZE):
        sl = slice(_SIZE * j, _SIZE * (j + 1))
        if j % 2 == 0:
            view[:, sl] = gxn * gy[:, j:j + 1]
        else:
            t = pb - (_LO + _STEP * j)
            view[:, sl] = gxn * jnp.exp(t * t * inv)

    pltpu.make_async_copy(
        scratch.at[buf],
        out_hbm.at[pl.ds(k * _ROWS, _ROWS), :],
        sems.at[buf],
    ).start()

    @pl.when(k == _NBLK - 1)
    def _():
        for off in range(_NBUF - 1, -1, -1):
            kk = _NBLK - 1 - off
            pltpu.make_async_copy(
                scratch.at[jax.lax.rem(jnp.int32(kk), _NBUF)],
                out_hbm.at[pl.ds(kk * _ROWS, _ROWS), :],
                sems.at[jax.lax.rem(jnp.int32(kk), _NBUF)],
            ).wait()


def kernel(inp, sg):
    B, P, _ = inp.shape
    n = B * P
    pts = inp.reshape(n, 2)
    out = pl.pallas_call(
        _rbf_body,
        out_shape=jax.ShapeDtypeStruct((n, _SIZE * _SIZE), jnp.float32),
        grid=(_NBLK,),
        in_specs=[
            pl.BlockSpec(memory_space=pltpu.SMEM),
            pl.BlockSpec((_ROWS, 2), lambda i: (i, 0)),
        ],
        out_specs=pl.BlockSpec(memory_space=pl.ANY),
        scratch_shapes=[
            pltpu.VMEM((_NBUF, _ROWS, _SIZE * _SIZE), jnp.float32),
            pltpu.SemaphoreType.DMA((_NBUF,)),
        ],
        compiler_params=pltpu.CompilerParams(
            dimension_semantics=("arbitrary",),
            vmem_limit_bytes=56 * 1024 * 1024,
        ),
        name="rbf_splat",
    )(sg, pts)
    return out.reshape(B, P, _SIZE * _SIZE)
